# Initial kernel scaffold; baseline (speedup 1.0000x reference)
import jax
import jax.numpy as jnp
from jax.experimental import pallas as pl


def kernel(sparse_indices, dense_features, emb_tables, gamma, beta, W, b):
    raise NotImplementedError("write your pallas kernel here")


# speedup vs baseline: 8.0607x; 8.0607x over previous
import functools

import jax
import jax.numpy as jnp
from jax import lax
from jax.experimental import pallas as pl
from jax.experimental.pallas import tpu as pltpu
from jax.experimental.pallas import tpu_sc as plsc

B = 16384
F = 26
V = 100000
D = 16
NDENSE = 13
EPS = 1e-3

NC = 2
NS = 16
NW = NC * NS
CB = B // NW
CHUNK_B = 128
ROWS_PER_CHUNK = CHUNK_B * F
IDX_W = 128
NDMA = ROWS_PER_CHUNK // IDX_W
NCHUNK = CB // CHUNK_B
IDX_ROWS_PER_WORKER = CB * F // IDX_W


def _sc_body(table_hbm, idx_hbm, wemb_hbm, out_hbm, idx_v, rows_v, w_v, out_v, sem):
    wid = lax.axis_index("s") * NC + lax.axis_index("c")
    base_b = wid * CB

    pltpu.sync_copy(wemb_hbm, w_v)
    wvecs = [w_v[j] for j in range(F)]

    for ci in range(NCHUNK):
        pltpu.sync_copy(idx_hbm.at[wid * NCHUNK + ci], idx_v)
        copies = [
            pltpu.async_copy(
                table_hbm.at[idx_v.at[j]],
                rows_v.at[pl.ds(j * IDX_W, IDX_W)],
                sem,
            )
            for j in range(NDMA)
        ]
        for c in copies:
            c.wait()

        def elem_body(i, _):
            r0 = i * F
            acc = wvecs[0] * rows_v[r0]
            for j in range(1, F):
                acc = acc + wvecs[j] * rows_v[r0 + j]
            out_v[i] = acc
            return 0

        lax.fori_loop(0, CHUNK_B, elem_body, 0)
        pltpu.sync_copy(
            out_v, out_hbm.at[pl.ds(base_b + ci * CHUNK_B, CHUNK_B)]
        )


@jax.jit
def _sc_emb_dot(flat_table, idx2d, wemb):
    mesh = plsc.VectorSubcoreMesh(core_axis_name="c", subcore_axis_name="s")
    return pl.kernel(
        _sc_body,
        out_type=jax.ShapeDtypeStruct((B, D), jnp.float32),
        mesh=mesh,
        scratch_types=[
            pltpu.VMEM((NDMA, IDX_W), jnp.int32),
            pltpu.VMEM((ROWS_PER_CHUNK, D), jnp.float32),
            pltpu.VMEM((F, D), jnp.float32),
            pltpu.VMEM((CHUNK_B, D), jnp.float32),
            pltpu.SemaphoreType.DMA,
        ],
        compiler_params=pltpu.CompilerParams(use_tc_tiling_on_sc=False),
    )(flat_table, idx2d, wemb)


def _tc_body(x_ref, p_ref, gamma_ref, beta_ref, wd_ref, b_ref, out_ref):
    x = x_ref[:]
    mean = jnp.mean(x, axis=0, keepdims=True)
    var = jnp.mean((x - mean) ** 2, axis=0, keepdims=True)
    rstd = lax.rsqrt(var + EPS)
    alpha = gamma_ref[:] * wd_ref[:] * rstd
    const = jnp.sum(beta_ref[:] * wd_ref[:] - alpha * mean) + b_ref[0, 0]
    d = jnp.sum(x * alpha, axis=1, keepdims=True)
    s = jnp.sum(p_ref[:], axis=1, keepdims=True)
    out_ref[:] = jax.nn.sigmoid(s + d + const)


@jax.jit
def _tc_finish(dense_features, partials, gamma, beta, wd, b):
    return pl.pallas_call(
        _tc_body,
        out_shape=jax.ShapeDtypeStruct((B, 1), jnp.float32),
    )(dense_features, partials, gamma, beta, wd, b)


def kernel(sparse_indices, dense_features, emb_tables, gamma, beta, W, b):
    flat_table = emb_tables.reshape(F * V, D)
    offsets = (jnp.arange(F, dtype=jnp.int32) * V)[None, :]
    idx2d = (sparse_indices + offsets).reshape(NW * NCHUNK, NDMA, IDX_W)
    wemb = W[: F * D, 0].reshape(F, D)
    partials = _sc_emb_dot(flat_table, idx2d, wemb)
    return _tc_finish(
        dense_features,
        partials,
        gamma.reshape(1, NDENSE),
        beta.reshape(1, NDENSE),
        W[F * D :, 0].reshape(1, NDENSE),
        b.reshape(1, 1),
    )


# speedup vs baseline: 20.6512x; 2.5620x over previous
import jax
import jax.numpy as jnp
from jax import lax
from jax.experimental import pallas as pl
from jax.experimental.pallas import tpu as pltpu
from jax.experimental.pallas import tpu_sc as plsc

B = 16384
F = 26
V = 100000
D = 16
NDENSE = 13
EPS = 1e-3

NC = 2
NS = 16
NW = NC * NS

VC = 2000
UNITS = F * (V // VC)
UPW = (UNITS + NW - 1) // NW

CB = B // NW
CHUNK_B = 128
NCHUNK = CB // CHUNK_B


def _phase_a_body(tbl_hbm, w_hbm, t_hbm, buf, tout, w_v, sems):
    wid = lax.axis_index("s") * NC + lax.axis_index("c")
    pltpu.sync_copy(w_hbm, w_v)

    cpv = V // VC

    def unit(i):
        u = jnp.minimum(i * NW + wid, UNITS - 1)
        return u // cpv, lax.rem(u, cpv) * VC

    def start(i, slot):
        f, v0 = unit(i)
        pltpu.async_copy(
            tbl_hbm.at[f, :, pl.ds(v0, VC)], buf.at[slot], sems.at[slot]
        )

    start(0, 0)
    start(1, 1)

    def loop(i, _):
        slot = lax.rem(i, 2)
        f, v0 = unit(i)
        pltpu.make_async_copy(
            tbl_hbm.at[f, :, pl.ds(v0, VC)], buf.at[slot], sems.at[slot]
        ).wait()
        wrow = w_v[f]
        ws = [wrow[d] for d in range(D)]

        def grp(g, _):
            sl = pl.ds(g * 16, 16)
            acc = buf[slot, 0, sl] * ws[0]
            for d in range(1, D):
                acc = acc + buf[slot, d, sl] * ws[d]
            tout[sl] = acc
            return 0

        lax.fori_loop(0, VC // 16, grp, 0)
        pltpu.sync_copy(tout, t_hbm.at[pl.ds(f * V + v0, VC)])

        @pl.when(i + 2 < UPW)
        def _():
            start_i = i + 2
            fs, vs = unit(start_i)
            pltpu.async_copy(
                tbl_hbm.at[fs, :, pl.ds(vs, VC)], buf.at[slot], sems.at[slot]
            )

        return 0

    lax.fori_loop(0, UPW, loop, 0)


def _phase_b_body(t_hbm, idx_hbm, s_hbm, idx_v, g_v, out_v, sem):
    wid = lax.axis_index("s") * NC + lax.axis_index("c")

    for ci in range(NCHUNK):
        pltpu.sync_copy(idx_hbm.at[wid * NCHUNK + ci], idx_v)
        copies = [
            pltpu.async_copy(t_hbm.at[idx_v.at[f]], g_v.at[f], sem)
            for f in range(F)
        ]
        for c in copies:
            c.wait()

        def grp(g, _):
            sl = pl.ds(g * 16, 16)
            acc = g_v[0, sl]
            for f in range(1, F):
                acc = acc + g_v[f, sl]
            out_v[sl] = acc
            return 0

        lax.fori_loop(0, CHUNK_B // 16, grp, 0)
        pltpu.sync_copy(
            out_v, s_hbm.at[pl.ds(wid * CB + ci * CHUNK_B, CHUNK_B)]
        )


@jax.jit
def _sc_pipeline(tbl_t, wemb, idx3d):
    mesh = plsc.VectorSubcoreMesh(core_axis_name="c", subcore_axis_name="s")
    params = pltpu.CompilerParams(use_tc_tiling_on_sc=False)
    t = pl.kernel(
        _phase_a_body,
        out_type=jax.ShapeDtypeStruct((F * V,), jnp.float32),
        mesh=mesh,
        scratch_types=[
            pltpu.VMEM((2, D, VC), jnp.float32),
            pltpu.VMEM((VC,), jnp.float32),
            pltpu.VMEM((F, D), jnp.float32),
            pltpu.SemaphoreType.DMA((2,)),
        ],
        compiler_params=params,
    )(tbl_t, wemb)
    return pl.kernel(
        _phase_b_body,
        out_type=jax.ShapeDtypeStruct((B,), jnp.float32),
        mesh=mesh,
        scratch_types=[
            pltpu.VMEM((F, CHUNK_B), jnp.int32),
            pltpu.VMEM((F, CHUNK_B), jnp.float32),
            pltpu.VMEM((CHUNK_B,), jnp.float32),
            pltpu.SemaphoreType.DMA,
        ],
        compiler_params=params,
    )(t, idx3d)


def _tc_body(x_ref, s_ref, gamma_ref, beta_ref, wd_ref, b_ref, out_ref):
    x = x_ref[:]
    mean = jnp.mean(x, axis=0, keepdims=True)
    var = jnp.mean((x - mean) ** 2, axis=0, keepdims=True)
    rstd = lax.rsqrt(var + EPS)
    alpha = gamma_ref[:] * wd_ref[:] * rstd
    const = jnp.sum(beta_ref[:] * wd_ref[:] - alpha * mean) + b_ref[0, 0]
    d = jnp.sum(x * alpha, axis=1, keepdims=True)
    out_ref[:] = jax.nn.sigmoid(s_ref[:] + d + const)


@jax.jit
def _tc_finish(dense_features, s_emb, gamma, beta, wd, b):
    return pl.pallas_call(
        _tc_body,
        out_shape=jax.ShapeDtypeStruct((B, 1), jnp.float32),
    )(dense_features, s_emb, gamma, beta, wd, b)


def kernel(sparse_indices, dense_features, emb_tables, gamma, beta, W, b):
    tbl_t = jnp.transpose(emb_tables, (0, 2, 1))
    wemb = W[: F * D, 0].reshape(F, D)
    si3 = sparse_indices.reshape(B // CHUNK_B, CHUNK_B, F)
    idx3d = jnp.transpose(si3, (0, 2, 1)) + (
        jnp.arange(F, dtype=jnp.int32) * V
    )[None, :, None]
    s_emb = _sc_pipeline(tbl_t, wemb, idx3d).reshape(B, 1)
    return _tc_finish(
        dense_features,
        s_emb,
        gamma.reshape(1, NDENSE),
        beta.reshape(1, NDENSE),
        W[F * D :, 0].reshape(1, NDENSE),
        b.reshape(1, 1),
    )


# speedup vs baseline: 26.1195x; 1.2648x over previous
import jax
import jax.numpy as jnp
from jax import lax
from jax.experimental import pallas as pl
from jax.experimental.pallas import tpu as pltpu
from jax.experimental.pallas import tpu_sc as plsc

B = 16384
F = 26
V = 100000
D = 16
NDENSE = 13
EPS = 1e-3

NC = 2
NS = 16
NW = NC * NS

CB = B // NW
CHUNK_B = 128
NCHUNK = CB // CHUNK_B


VB = 12544


def _ta_body(x_ref, w_ref, o_ref):
    f = pl.program_id(0)
    acc = x_ref[0, 0:1, :] * w_ref[f, 0]
    for d in range(1, D):
        acc = acc + x_ref[0, d : d + 1, :] * w_ref[f, d]
    o_ref[0, 0, :] = acc[0]


@jax.jit
def _tc_weighted_table(tbl_t, wemb):
    nvb = (V + VB - 1) // VB
    return pl.pallas_call(
        _ta_body,
        grid=(F, nvb),
        in_specs=[
            pl.BlockSpec((1, D, VB), lambda f, v: (f, 0, v)),
            pl.BlockSpec(memory_space=pltpu.SMEM),
        ],
        out_specs=pl.BlockSpec((1, 1, VB), lambda f, v: (f, 0, v)),
        out_shape=jax.ShapeDtypeStruct((F, 1, V), jnp.float32),
    )(tbl_t, wemb)


def _phase_b_body(t_hbm, idx_hbm, s_hbm, idx_v, g_v, out_v, sem):
    wid = lax.axis_index("s") * NC + lax.axis_index("c")

    for ci in range(NCHUNK):
        pltpu.sync_copy(idx_hbm.at[wid * NCHUNK + ci], idx_v)
        copies = [
            pltpu.async_copy(t_hbm.at[idx_v.at[f]], g_v.at[f], sem)
            for f in range(F)
        ]
        for c in copies:
            c.wait()

        def grp(g, _):
            sl = pl.ds(g * 16, 16)
            acc = g_v[0, sl]
            for f in range(1, F):
                acc = acc + g_v[f, sl]
            out_v[sl] = acc
            return 0

        lax.fori_loop(0, CHUNK_B // 16, grp, 0)
        pltpu.sync_copy(
            out_v, s_hbm.at[pl.ds(wid * CB + ci * CHUNK_B, CHUNK_B)]
        )


@jax.jit
def _sc_gather(t, idx3d):
    mesh = plsc.VectorSubcoreMesh(core_axis_name="c", subcore_axis_name="s")
    params = pltpu.CompilerParams(use_tc_tiling_on_sc=False)
    return pl.kernel(
        _phase_b_body,
        out_type=jax.ShapeDtypeStruct((B,), jnp.float32),
        mesh=mesh,
        scratch_types=[
            pltpu.VMEM((F, CHUNK_B), jnp.int32),
            pltpu.VMEM((F, CHUNK_B), jnp.float32),
            pltpu.VMEM((CHUNK_B,), jnp.float32),
            pltpu.SemaphoreType.DMA,
        ],
        compiler_params=params,
    )(t, idx3d)


def _tc_body(x_ref, s_ref, gamma_ref, beta_ref, wd_ref, b_ref, out_ref):
    x = x_ref[:]
    mean = jnp.mean(x, axis=0, keepdims=True)
    var = jnp.mean((x - mean) ** 2, axis=0, keepdims=True)
    rstd = lax.rsqrt(var + EPS)
    alpha = gamma_ref[:] * wd_ref[:] * rstd
    const = jnp.sum(beta_ref[:] * wd_ref[:] - alpha * mean) + b_ref[0, 0]
    d = jnp.sum(x * alpha, axis=1, keepdims=True)
    out_ref[:] = jax.nn.sigmoid(s_ref[:] + d + const)


@jax.jit
def _tc_finish(dense_features, s_emb, gamma, beta, wd, b):
    return pl.pallas_call(
        _tc_body,
        out_shape=jax.ShapeDtypeStruct((B, 1), jnp.float32),
    )(dense_features, s_emb, gamma, beta, wd, b)


def kernel(sparse_indices, dense_features, emb_tables, gamma, beta, W, b):
    tbl_t = jnp.transpose(emb_tables, (0, 2, 1))
    wemb = W[: F * D, 0].reshape(F, D)
    si3 = sparse_indices.reshape(B // CHUNK_B, CHUNK_B, F)
    idx3d = jnp.transpose(si3, (0, 2, 1)) + (
        jnp.arange(F, dtype=jnp.int32) * V
    )[None, :, None]
    t = _tc_weighted_table(tbl_t, wemb).reshape(F * V)
    s_emb = _sc_gather(t, idx3d).reshape(B, 1)
    return _tc_finish(
        dense_features,
        s_emb,
        gamma.reshape(1, NDENSE),
        beta.reshape(1, NDENSE),
        W[F * D :, 0].reshape(1, NDENSE),
        b.reshape(1, 1),
    )


# speedup vs baseline: 38.8455x; 1.4872x over previous
import jax
import jax.numpy as jnp
from jax import lax
from jax.experimental import pallas as pl
from jax.experimental.pallas import tpu as pltpu
from jax.experimental.pallas import tpu_sc as plsc

B = 16384
F = 26
V = 100000
D = 16
NDENSE = 13
EPS = 1e-3

NC = 2
NS = 16
NW = NC * NS

CB = B // NW
CHUNK_B = 128
NCHUNK = CB // CHUNK_B


VB = 13312
NVB = 8
VP = VB * NVB


def _ta_body(x_ref, w_ref, o_ref):
    o_ref[:] = jnp.sum(x_ref[0] * w_ref[0], axis=0)


@jax.jit
def _tc_weighted_table(tbl_t, wemb3):
    return pl.pallas_call(
        _ta_body,
        grid=(F, NVB),
        in_specs=[
            pl.BlockSpec((1, D, VB), lambda f, v: (f, 0, v)),
            pl.BlockSpec((1, D, 1), lambda f, v: (f, 0, 0)),
        ],
        out_specs=pl.BlockSpec((VB,), lambda f, v: (f * NVB + v,)),
        out_shape=jax.ShapeDtypeStruct((F * VP,), jnp.float32),
    )(tbl_t, wemb3)


def _phase_b_body(t_hbm, idx_hbm, s_hbm, idx_v, g_v, out_v, sem):
    wid = lax.axis_index("s") * NC + lax.axis_index("c")

    for ci in range(NCHUNK):
        pltpu.sync_copy(idx_hbm.at[wid * NCHUNK + ci], idx_v)
        copies = [
            pltpu.async_copy(t_hbm.at[idx_v.at[f]], g_v.at[f], sem)
            for f in range(F)
        ]
        for c in copies:
            c.wait()

        def grp(g, _):
            sl = pl.ds(g * 16, 16)
            acc = g_v[0, sl]
            for f in range(1, F):
                acc = acc + g_v[f, sl]
            out_v[sl] = acc
            return 0

        lax.fori_loop(0, CHUNK_B // 16, grp, 0)
        pltpu.sync_copy(
            out_v, s_hbm.at[pl.ds(wid * CB + ci * CHUNK_B, CHUNK_B)]
        )


@jax.jit
def _sc_gather(t, idx3d):
    mesh = plsc.VectorSubcoreMesh(core_axis_name="c", subcore_axis_name="s")
    params = pltpu.CompilerParams(use_tc_tiling_on_sc=False)
    return pl.kernel(
        _phase_b_body,
        out_type=jax.ShapeDtypeStruct((B,), jnp.float32),
        mesh=mesh,
        scratch_types=[
            pltpu.VMEM((F, CHUNK_B), jnp.int32),
            pltpu.VMEM((F, CHUNK_B), jnp.float32),
            pltpu.VMEM((CHUNK_B,), jnp.float32),
            pltpu.SemaphoreType.DMA,
        ],
        compiler_params=params,
    )(t, idx3d)


def _tc_body(x_ref, s_ref, gamma_ref, beta_ref, wd_ref, b_ref, out_ref):
    x = x_ref[:]
    mean = jnp.mean(x, axis=0, keepdims=True)
    var = jnp.mean((x - mean) ** 2, axis=0, keepdims=True)
    rstd = lax.rsqrt(var + EPS)
    alpha = gamma_ref[:] * wd_ref[:] * rstd
    const = jnp.sum(beta_ref[:] * wd_ref[:] - alpha * mean) + b_ref[0, 0]
    d = jnp.sum(x * alpha, axis=1, keepdims=True)
    out_ref[:] = jax.nn.sigmoid(s_ref[:] + d + const)


@jax.jit
def _tc_finish(dense_features, s_emb, gamma, beta, wd, b):
    return pl.pallas_call(
        _tc_body,
        out_shape=jax.ShapeDtypeStruct((B, 1), jnp.float32),
    )(dense_features, s_emb, gamma, beta, wd, b)


def kernel(sparse_indices, dense_features, emb_tables, gamma, beta, W, b):
    tbl_t = jnp.transpose(emb_tables, (0, 2, 1))
    wemb3 = W[: F * D, 0].reshape(F, D, 1)
    si3 = sparse_indices.reshape(B // CHUNK_B, CHUNK_B, F)
    idx3d = jnp.transpose(si3, (0, 2, 1)) + (
        jnp.arange(F, dtype=jnp.int32) * VP
    )[None, :, None]
    t = _tc_weighted_table(tbl_t, wemb3)
    s_emb = _sc_gather(t, idx3d).reshape(B, 1)
    return _tc_finish(
        dense_features,
        s_emb,
        gamma.reshape(1, NDENSE),
        beta.reshape(1, NDENSE),
        W[F * D :, 0].reshape(1, NDENSE),
        b.reshape(1, 1),
    )


# speedup vs baseline: 52.6689x; 1.3559x over previous
import jax
import jax.numpy as jnp
from jax import lax
from jax.experimental import pallas as pl
from jax.experimental.pallas import tpu as pltpu
from jax.experimental.pallas import tpu_sc as plsc

B = 16384
F = 26
V = 100000
D = 16
NDENSE = 13
EPS = 1e-3

NC = 2
NS = 16
NW = NC * NS

CB = B // NW
CHUNK_B = 128
NCHUNK = CB // CHUNK_B


VB = 13312
NVB = 8
VP = VB * NVB

KSC = 14
FTC = F - KSC
VPS = 102400
VCS = 1408
CPF = 71
UNITS_S = KSC * CPF
UPWS = (UNITS_S + NW - 1) // NW
TAIL_OFF = 99328


def _ta_body(x_ref, w_ref, o_ref):
    o_ref[:] = jnp.sum(x_ref[0] * w_ref[0], axis=0)


@jax.jit
def _tc_weighted_table(tbl_t, wemb3):
    return pl.pallas_call(
        _ta_body,
        grid=(FTC, NVB),
        in_specs=[
            pl.BlockSpec((1, D, VB), lambda f, v: (f + KSC, 0, v)),
            pl.BlockSpec((1, D, 1), lambda f, v: (f + KSC, 0, 0)),
        ],
        out_specs=pl.BlockSpec((VB,), lambda f, v: (f * NVB + v,)),
        out_shape=jax.ShapeDtypeStruct((FTC * VP,), jnp.float32),
    )(tbl_t, wemb3)


def _sa_body(tbl_hbm, w_hbm, t_hbm, buf, tout, w_v, sems):
    wid = lax.axis_index("s") * NC + lax.axis_index("c")
    pltpu.sync_copy(w_hbm, w_v)

    def unit(i):
        u = jnp.minimum(i * NW + wid, UNITS_S - 1)
        return u // CPF, lax.rem(u, CPF) * VCS

    def start(i, slot):
        f, v0 = unit(i)
        pltpu.async_copy(
            tbl_hbm.at[f, :, pl.ds(v0, VCS)], buf.at[slot], sems.at[slot]
        )

    start(0, 0)
    start(1, 1)

    def loop(i, _):
        slot = lax.rem(i, 2)
        f, v0 = unit(i)
        pltpu.make_async_copy(
            tbl_hbm.at[f, :, pl.ds(v0, VCS)], buf.at[slot], sems.at[slot]
        ).wait()
        wrow = w_v[pl.ds(f * D, D)]
        ws = [wrow[d] for d in range(D)]

        def grp(g, _):
            sl = pl.ds(g * 16, 16)
            acc = buf[slot, 0, sl] * ws[0]
            for d in range(1, D):
                acc = acc + buf[slot, d, sl] * ws[d]
            tout[sl] = acc
            return 0

        lax.fori_loop(0, VCS // 16, grp, 0)
        pltpu.sync_copy(tout, t_hbm.at[pl.ds(f * VPS + v0, VCS)])

        @pl.when(i + 2 < UPWS)
        def _():
            fs, vs = unit(i + 2)
            pltpu.async_copy(
                tbl_hbm.at[fs, :, pl.ds(vs, VCS)], buf.at[slot], sems.at[slot]
            )

        return 0

    lax.fori_loop(0, UPWS, loop, 0)


@jax.jit
def _sc_table(tbl_t, wflat):
    mesh = plsc.VectorSubcoreMesh(core_axis_name="c", subcore_axis_name="s")
    return pl.kernel(
        _sa_body,
        out_type=jax.ShapeDtypeStruct((KSC * VPS,), jnp.float32),
        mesh=mesh,
        scratch_types=[
            pltpu.VMEM((2, D, VCS), jnp.float32),
            pltpu.VMEM((VCS,), jnp.float32),
            pltpu.VMEM((F * D,), jnp.float32),
            pltpu.SemaphoreType.DMA((2,)),
        ],
        compiler_params=pltpu.CompilerParams(use_tc_tiling_on_sc=True),
    )(tbl_t, wflat)


def _tail_body(x_ref, w_ref, t_ref, o_ref, acc_v, sem):
    f = pl.program_id(0)
    acc_v[:] = jnp.sum(x_ref[0] * w_ref[0], axis=0)
    pltpu.async_copy(acc_v, o_ref.at[pl.ds(f * VPS + TAIL_OFF, 1024)], sem).wait()


@jax.jit
def _tc_tail(tbl_t, wemb3, t_sc):
    return pl.pallas_call(
        _tail_body,
        grid=(KSC,),
        in_specs=[
            pl.BlockSpec((1, D, 1024), lambda f: (f, 0, TAIL_OFF // 1024)),
            pl.BlockSpec((1, D, 1), lambda f: (f, 0, 0)),
            pl.BlockSpec(memory_space=pl.ANY),
        ],
        out_specs=pl.BlockSpec(memory_space=pl.ANY),
        out_shape=jax.ShapeDtypeStruct((KSC * VPS,), jnp.float32),
        scratch_shapes=[
            pltpu.VMEM((1024,), jnp.float32),
            pltpu.SemaphoreType.DMA,
        ],
        input_output_aliases={2: 0},
    )(tbl_t, wemb3, t_sc)


def _phase_b_body(ts_hbm, tt_hbm, idx_hbm, s_hbm, idx_v, g_v, out_v, sem):
    wid = lax.axis_index("s") * NC + lax.axis_index("c")

    for ci in range(NCHUNK):
        pltpu.sync_copy(idx_hbm.at[wid * NCHUNK + ci], idx_v)
        copies = [
            pltpu.async_copy(
                (ts_hbm if f < KSC else tt_hbm).at[idx_v.at[f]], g_v.at[f], sem
            )
            for f in range(F)
        ]
        for c in copies:
            c.wait()

        def grp(g, _):
            sl = pl.ds(g * 16, 16)
            acc = g_v[0, sl]
            for f in range(1, F):
                acc = acc + g_v[f, sl]
            out_v[sl] = acc
            return 0

        lax.fori_loop(0, CHUNK_B // 16, grp, 0)
        pltpu.sync_copy(
            out_v, s_hbm.at[pl.ds(wid * CB + ci * CHUNK_B, CHUNK_B)]
        )


@jax.jit
def _sc_gather(t_sc, t_tc, idx3d):
    mesh = plsc.VectorSubcoreMesh(core_axis_name="c", subcore_axis_name="s")
    params = pltpu.CompilerParams(use_tc_tiling_on_sc=False)
    return pl.kernel(
        _phase_b_body,
        out_type=jax.ShapeDtypeStruct((B,), jnp.float32),
        mesh=mesh,
        scratch_types=[
            pltpu.VMEM((F, CHUNK_B), jnp.int32),
            pltpu.VMEM((F, CHUNK_B), jnp.float32),
            pltpu.VMEM((CHUNK_B,), jnp.float32),
            pltpu.SemaphoreType.DMA,
        ],
        compiler_params=params,
    )(t_sc, t_tc, idx3d)


def _tc_body(x_ref, s_ref, gamma_ref, beta_ref, wd_ref, b_ref, out_ref):
    x = x_ref[:]
    mean = jnp.mean(x, axis=0, keepdims=True)
    var = jnp.mean((x - mean) ** 2, axis=0, keepdims=True)
    rstd = lax.rsqrt(var + EPS)
    alpha = gamma_ref[:] * wd_ref[:] * rstd
    const = jnp.sum(beta_ref[:] * wd_ref[:] - alpha * mean) + b_ref[0, 0]
    d = jnp.sum(x * alpha, axis=1, keepdims=True)
    out_ref[:] = jax.nn.sigmoid(s_ref[:] + d + const)


@jax.jit
def _tc_finish(dense_features, s_emb, gamma, beta, wd, b):
    return pl.pallas_call(
        _tc_body,
        out_shape=jax.ShapeDtypeStruct((B, 1), jnp.float32),
    )(dense_features, s_emb, gamma, beta, wd, b)


def kernel(sparse_indices, dense_features, emb_tables, gamma, beta, W, b):
    tbl_t = jnp.transpose(emb_tables, (0, 2, 1))
    wemb3 = W[: F * D, 0].reshape(F, D, 1)
    wflat = W[: F * D, 0]
    fidx = jnp.arange(F, dtype=jnp.int32)
    offs = jnp.where(fidx < KSC, fidx * VPS, (fidx - KSC) * VP)
    si3 = sparse_indices.reshape(B // CHUNK_B, CHUNK_B, F)
    idx3d = jnp.transpose(si3, (0, 2, 1)) + offs[None, :, None]
    t_sc = _tc_tail(tbl_t, wemb3, _sc_table(tbl_t, wflat))
    t_tc = _tc_weighted_table(tbl_t, wemb3)
    s_emb = _sc_gather(t_sc, t_tc, idx3d).reshape(B, 1)
    return _tc_finish(
        dense_features,
        s_emb,
        gamma.reshape(1, NDENSE),
        beta.reshape(1, NDENSE),
        W[F * D :, 0].reshape(1, NDENSE),
        b.reshape(1, 1),
    )


# speedup vs baseline: 62.8594x; 1.1935x over previous
import jax
import jax.numpy as jnp
from jax import lax
from jax.experimental import pallas as pl
from jax.experimental.pallas import tpu as pltpu
from jax.experimental.pallas import tpu_sc as plsc

B = 16384
F = 26
V = 100000
D = 16
NDENSE = 13
EPS = 1e-3

NC = 2
NS = 16
NW = NC * NS

CB = B // NW
CHUNK_B = 128
NCHUNK = CB // CHUNK_B


VB = 13312
NVB = 8
VP = VB * NVB

KSC = 15
FTC = F - KSC
VPS = 102400
VCS = 1408
CPF = 71
UNITS_S = KSC * CPF
UPWS = (UNITS_S + NW - 1) // NW
TAIL_OFF = 99328


def _ta_body(x_ref, w_ref, o_ref):
    o_ref[:] = jnp.sum(x_ref[0] * w_ref[0], axis=0)


@jax.jit
def _tc_weighted_table(tbl_t, wemb3):
    return pl.pallas_call(
        _ta_body,
        grid=(FTC, NVB),
        in_specs=[
            pl.BlockSpec((1, D, VB), lambda f, v: (f + KSC, 0, v)),
            pl.BlockSpec((1, D, 1), lambda f, v: (f + KSC, 0, 0)),
        ],
        out_specs=pl.BlockSpec((VB,), lambda f, v: (f * NVB + v,)),
        out_shape=jax.ShapeDtypeStruct((FTC * VP,), jnp.float32),
    )(tbl_t, wemb3)


def _sa_body(tbl_hbm, w_hbm, t_hbm, buf, tout, w_v, sems):
    wid = lax.axis_index("s") * NC + lax.axis_index("c")
    pltpu.sync_copy(w_hbm, w_v)

    def unit(i):
        u = jnp.minimum(i * NW + wid, UNITS_S - 1)
        return u // CPF, lax.rem(u, CPF) * VCS

    def start(i, slot):
        f, v0 = unit(i)
        pltpu.async_copy(
            tbl_hbm.at[f, :, pl.ds(v0, VCS)], buf.at[slot], sems.at[slot]
        )

    start(0, 0)
    start(1, 1)

    def loop(i, _):
        slot = lax.rem(i, 2)
        f, v0 = unit(i)
        pltpu.make_async_copy(
            tbl_hbm.at[f, :, pl.ds(v0, VCS)], buf.at[slot], sems.at[slot]
        ).wait()
        wrow = w_v[pl.ds(f * D, D)]
        ws = [wrow[d] for d in range(D)]

        def grp(g, _):
            sl = pl.ds(g * 16, 16)
            acc = buf[slot, 0, sl] * ws[0]
            for d in range(1, D):
                acc = acc + buf[slot, d, sl] * ws[d]
            tout[sl] = acc
            return 0

        lax.fori_loop(0, VCS // 16, grp, 0)
        pltpu.sync_copy(tout, t_hbm.at[pl.ds(f * VPS + v0, VCS)])

        @pl.when(i + 2 < UPWS)
        def _():
            fs, vs = unit(i + 2)
            pltpu.async_copy(
                tbl_hbm.at[fs, :, pl.ds(vs, VCS)], buf.at[slot], sems.at[slot]
            )

        return 0

    lax.fori_loop(0, UPWS, loop, 0)


@jax.jit
def _sc_table(tbl_t, wflat):
    mesh = plsc.VectorSubcoreMesh(core_axis_name="c", subcore_axis_name="s")
    return pl.kernel(
        _sa_body,
        out_type=jax.ShapeDtypeStruct((KSC * VPS,), jnp.float32),
        mesh=mesh,
        scratch_types=[
            pltpu.VMEM((2, D, VCS), jnp.float32),
            pltpu.VMEM((VCS,), jnp.float32),
            pltpu.VMEM((F * D,), jnp.float32),
            pltpu.SemaphoreType.DMA((2,)),
        ],
        compiler_params=pltpu.CompilerParams(use_tc_tiling_on_sc=True),
    )(tbl_t, wflat)


def _tail_body(x_ref, w_ref, t_ref, o_ref, acc_v, sem):
    for f in range(KSC):
        acc_v[pl.ds(f * 1024, 1024)] = jnp.sum(x_ref[f] * w_ref[f], axis=0)
    copies = [
        pltpu.async_copy(
            acc_v.at[pl.ds(f * 1024, 1024)],
            o_ref.at[pl.ds(f * VPS + TAIL_OFF, 1024)],
            sem,
        )
        for f in range(KSC)
    ]
    for c in copies:
        c.wait()


@jax.jit
def _tc_tail(tbl_t, wemb3, t_sc):
    return pl.pallas_call(
        _tail_body,
        grid=(1,),
        in_specs=[
            pl.BlockSpec((KSC, D, 1024), lambda i: (0, 0, TAIL_OFF // 1024)),
            pl.BlockSpec((KSC, D, 1), lambda i: (0, 0, 0)),
            pl.BlockSpec(memory_space=pl.ANY),
        ],
        out_specs=pl.BlockSpec(memory_space=pl.ANY),
        out_shape=jax.ShapeDtypeStruct((KSC * VPS,), jnp.float32),
        scratch_shapes=[
            pltpu.VMEM((KSC * 1024,), jnp.float32),
            pltpu.SemaphoreType.DMA,
        ],
        input_output_aliases={2: 0},
    )(tbl_t, wemb3, t_sc)


def _phase_b_body(
    ts_hbm, tt_hbm, idx_hbm, x_hbm, ac_hbm, out_hbm, idx_v, g_v, x_v, ac_v, out_v, sem
):
    wid = lax.axis_index("s") * NC + lax.axis_index("c")
    b0 = wid * CB

    pltpu.sync_copy(ac_hbm, ac_v)
    pltpu.sync_copy(idx_hbm.at[pl.ds(wid * NCHUNK, NCHUNK)], idx_v)
    pltpu.sync_copy(x_hbm.at[:, pl.ds(b0, CB)], x_v)
    copies = [
        pltpu.async_copy(
            (ts_hbm if f < KSC else tt_hbm).at[idx_v.at[ci, f]],
            g_v.at[ci, f],
            sem,
        )
        for ci in range(NCHUNK)
        for f in range(F)
    ]
    for c in copies:
        c.wait()

    a_row = ac_v[0]
    als = [a_row[j] for j in range(NDENSE)]
    cst = ac_v[1][0]

    def grp(g, _):
        ci = g // (CHUNK_B // 16)
        sl = pl.ds(lax.rem(g, CHUNK_B // 16) * 16, 16)
        gsl = pl.ds(g * 16, 16)
        z = g_v[ci, 0, sl] + cst
        for f in range(1, F):
            z = z + g_v[ci, f, sl]
        for j in range(NDENSE):
            z = z + x_v[j, gsl] * als[j]
        out_v[gsl] = 1.0 / (1.0 + jnp.exp(-z))
        return 0

    lax.fori_loop(0, CB // 16, grp, 0)
    pltpu.sync_copy(out_v, out_hbm.at[pl.ds(b0, CB)])


@jax.jit
def _sc_gather(t_sc, t_tc, idx3d, x_t, alphac):
    mesh = plsc.VectorSubcoreMesh(core_axis_name="c", subcore_axis_name="s")
    params = pltpu.CompilerParams(use_tc_tiling_on_sc=False)
    return pl.kernel(
        _phase_b_body,
        out_type=jax.ShapeDtypeStruct((B,), jnp.float32),
        mesh=mesh,
        scratch_types=[
            pltpu.VMEM((NCHUNK, F, CHUNK_B), jnp.int32),
            pltpu.VMEM((NCHUNK, F, CHUNK_B), jnp.float32),
            pltpu.VMEM((NDENSE, CB), jnp.float32),
            pltpu.VMEM((2, 16), jnp.float32),
            pltpu.VMEM((CB,), jnp.float32),
            pltpu.SemaphoreType.DMA,
        ],
        compiler_params=params,
    )(t_sc, t_tc, idx3d, x_t, alphac)


def _stats_body(x_ref, gamma_ref, beta_ref, wd_ref, b_ref, out_ref):
    x = x_ref[:]
    mean = jnp.mean(x, axis=0, keepdims=True)
    var = jnp.mean((x - mean) ** 2, axis=0, keepdims=True)
    rstd = lax.rsqrt(var + EPS)
    alpha = gamma_ref[:] * wd_ref[:] * rstd
    const = jnp.sum(beta_ref[:] * wd_ref[:] - alpha * mean) + b_ref[0, 0]
    out_ref[0:1, 0:NDENSE] = alpha
    out_ref[0:1, NDENSE:16] = jnp.zeros((1, 16 - NDENSE), jnp.float32)
    out_ref[1:2, :] = jnp.full((1, 16), const, jnp.float32)


@jax.jit
def _tc_stats(dense_features, gamma, beta, wd, b):
    return pl.pallas_call(
        _stats_body,
        out_shape=jax.ShapeDtypeStruct((2, 16), jnp.float32),
    )(dense_features, gamma, beta, wd, b)


def kernel(sparse_indices, dense_features, emb_tables, gamma, beta, W, b):
    tbl_t = jnp.transpose(emb_tables, (0, 2, 1))
    wemb3 = W[: F * D, 0].reshape(F, D, 1)
    wflat = W[: F * D, 0]
    fidx = jnp.arange(F, dtype=jnp.int32)
    offs = jnp.where(fidx < KSC, fidx * VPS, (fidx - KSC) * VP)
    si3 = sparse_indices.reshape(B // CHUNK_B, CHUNK_B, F)
    idx3d = jnp.transpose(si3, (0, 2, 1)) + offs[None, :, None]
    t_sc = _tc_tail(tbl_t, wemb3, _sc_table(tbl_t, wflat))
    t_tc = _tc_weighted_table(tbl_t, wemb3)
    alphac = _tc_stats(
        dense_features,
        gamma.reshape(1, NDENSE),
        beta.reshape(1, NDENSE),
        W[F * D :, 0].reshape(1, NDENSE),
        b.reshape(1, 1),
    )
    x_t = jnp.transpose(dense_features, (1, 0))
    return _sc_gather(t_sc, t_tc, idx3d, x_t, alphac).reshape(B, 1)


# speedup vs baseline: 69.2900x; 1.1023x over previous
import jax
import jax.numpy as jnp
from jax import lax
from jax.experimental import pallas as pl
from jax.experimental.pallas import tpu as pltpu
from jax.experimental.pallas import tpu_sc as plsc

B = 16384
F = 26
V = 100000
D = 16
NDENSE = 13
EPS = 1e-3

NC = 2
NS = 16
NW = NC * NS

CB = B // NW
CHUNK_B = 128
NCHUNK = CB // CHUNK_B


VB = 13312
NVB = 8
VP = VB * NVB

KSC = 15
FTC = F - KSC
VPS = 102400
VCS = 1408
CPF = 71
UNITS_S = KSC * CPF
UPWS = (UNITS_S + NW - 1) // NW
TAIL_OFF = 99328


def _ta_body(x_ref, w_ref, o_ref):
    o_ref[:] = jnp.sum(x_ref[0] * w_ref[0], axis=0)


@jax.jit
def _tc_weighted_table(tbl_t, wemb3):
    return pl.pallas_call(
        _ta_body,
        grid=(FTC, NVB),
        in_specs=[
            pl.BlockSpec((1, D, VB), lambda f, v: (f + KSC, 0, v)),
            pl.BlockSpec((1, D, 1), lambda f, v: (f + KSC, 0, 0)),
        ],
        out_specs=pl.BlockSpec((VB,), lambda f, v: (f * NVB + v,)),
        out_shape=jax.ShapeDtypeStruct((FTC * VP,), jnp.float32),
    )(tbl_t, wemb3)


def _sa_body(tbl_hbm, w_hbm, t_hbm, buf, tout, w_v, sems):
    wid = lax.axis_index("s") * NC + lax.axis_index("c")
    pltpu.sync_copy(w_hbm, w_v)

    def unit(i):
        u = jnp.minimum(i * NW + wid, UNITS_S - 1)
        return u // CPF, lax.rem(u, CPF) * VCS

    def start(i, slot):
        f, v0 = unit(i)
        pltpu.async_copy(
            tbl_hbm.at[f, :, pl.ds(v0, VCS)], buf.at[slot], sems.at[slot]
        )

    start(0, 0)
    start(1, 1)

    def loop(i, _):
        slot = lax.rem(i, 2)
        f, v0 = unit(i)
        pltpu.make_async_copy(
            tbl_hbm.at[f, :, pl.ds(v0, VCS)], buf.at[slot], sems.at[slot]
        ).wait()
        wrow = w_v[pl.ds(f * D, D)]
        ws = [wrow[d] for d in range(D)]

        def grp(g, _):
            sl = pl.ds(g * 16, 16)
            acc = buf[slot, 0, sl] * ws[0]
            for d in range(1, D):
                acc = acc + buf[slot, d, sl] * ws[d]
            tout[sl] = acc
            return 0

        lax.fori_loop(0, VCS // 16, grp, 0)
        pltpu.sync_copy(tout, t_hbm.at[pl.ds(f * VPS + v0, VCS)])

        @pl.when(i + 2 < UPWS)
        def _():
            fs, vs = unit(i + 2)
            pltpu.async_copy(
                tbl_hbm.at[fs, :, pl.ds(vs, VCS)], buf.at[slot], sems.at[slot]
            )

        return 0

    lax.fori_loop(0, UPWS, loop, 0)


@jax.jit
def _sc_table(tbl_t, wflat):
    mesh = plsc.VectorSubcoreMesh(core_axis_name="c", subcore_axis_name="s")
    return pl.kernel(
        _sa_body,
        out_type=jax.ShapeDtypeStruct((KSC * VPS,), jnp.float32),
        mesh=mesh,
        scratch_types=[
            pltpu.VMEM((2, D, VCS), jnp.float32),
            pltpu.VMEM((VCS,), jnp.float32),
            pltpu.VMEM((F * D,), jnp.float32),
            pltpu.SemaphoreType.DMA((2,)),
        ],
        compiler_params=pltpu.CompilerParams(use_tc_tiling_on_sc=True),
    )(tbl_t, wflat)


def _tail_body(x_ref, w_ref, t_ref, o_ref, acc_v, sem):
    for f in range(KSC):
        acc_v[pl.ds(f * 1024, 1024)] = jnp.sum(x_ref[f] * w_ref[f], axis=0)
    copies = [
        pltpu.async_copy(
            acc_v.at[pl.ds(f * 1024, 1024)],
            o_ref.at[pl.ds(f * VPS + TAIL_OFF, 1024)],
            sem,
        )
        for f in range(KSC)
    ]
    for c in copies:
        c.wait()


@jax.jit
def _tc_tail(tbl_t, wemb3, t_sc):
    return pl.pallas_call(
        _tail_body,
        grid=(1,),
        in_specs=[
            pl.BlockSpec((KSC, D, 1024), lambda i: (0, 0, TAIL_OFF // 1024)),
            pl.BlockSpec((KSC, D, 1), lambda i: (0, 0, 0)),
            pl.BlockSpec(memory_space=pl.ANY),
        ],
        out_specs=pl.BlockSpec(memory_space=pl.ANY),
        out_shape=jax.ShapeDtypeStruct((KSC * VPS,), jnp.float32),
        scratch_shapes=[
            pltpu.VMEM((KSC * 1024,), jnp.float32),
            pltpu.SemaphoreType.DMA,
        ],
        input_output_aliases={2: 0},
    )(tbl_t, wemb3, t_sc)


def _phase_b_body(
    ts_hbm, tt_hbm, idx_hbm, d_hbm, out_hbm, idx_v, g_v, d_v, out_v, sem
):
    wid = lax.axis_index("s") * NC + lax.axis_index("c")
    b0 = wid * CB

    pltpu.sync_copy(idx_hbm.at[pl.ds(wid * NCHUNK, NCHUNK)], idx_v)
    pltpu.sync_copy(d_hbm.at[pl.ds(b0, CB)], d_v)
    copies = [
        pltpu.async_copy(
            (ts_hbm if f < KSC else tt_hbm).at[idx_v.at[ci, f]],
            g_v.at[ci, f],
            sem,
        )
        for ci in range(NCHUNK)
        for f in range(F)
    ]
    for c in copies:
        c.wait()

    def grp(g, _):
        ci = g // (CHUNK_B // 16)
        sl = pl.ds(lax.rem(g, CHUNK_B // 16) * 16, 16)
        gsl = pl.ds(g * 16, 16)
        z = g_v[ci, 0, sl] + d_v[gsl]
        for f in range(1, F):
            z = z + g_v[ci, f, sl]
        out_v[gsl] = 1.0 / (1.0 + jnp.exp(-z))
        return 0

    lax.fori_loop(0, CB // 16, grp, 0)
    pltpu.sync_copy(out_v, out_hbm.at[pl.ds(b0, CB)])


@jax.jit
def _sc_gather(t_sc, t_tc, idx3d, d):
    mesh = plsc.VectorSubcoreMesh(core_axis_name="c", subcore_axis_name="s")
    params = pltpu.CompilerParams(use_tc_tiling_on_sc=False)
    return pl.kernel(
        _phase_b_body,
        out_type=jax.ShapeDtypeStruct((B,), jnp.float32),
        mesh=mesh,
        scratch_types=[
            pltpu.VMEM((NCHUNK, F, CHUNK_B), jnp.int32),
            pltpu.VMEM((NCHUNK, F, CHUNK_B), jnp.float32),
            pltpu.VMEM((CB,), jnp.float32),
            pltpu.VMEM((CB,), jnp.float32),
            pltpu.SemaphoreType.DMA,
        ],
        compiler_params=params,
    )(t_sc, t_tc, idx3d, d)


def _stats_body(xt_ref, gamma_ref, beta_ref, wd_ref, b_ref, out_ref):
    x = xt_ref[:]
    mean = jnp.mean(x, axis=1, keepdims=True)
    var = jnp.mean((x - mean) ** 2, axis=1, keepdims=True)
    alpha = gamma_ref[:] * wd_ref[:] * lax.rsqrt(var + EPS)
    const = jnp.sum(beta_ref[:] * wd_ref[:]) - jnp.sum(alpha * mean) + b_ref[0, 0]
    out_ref[:] = jnp.sum(x * alpha, axis=0) + const


@jax.jit
def _tc_stats(x_t, gamma, beta, wd, b):
    return pl.pallas_call(
        _stats_body,
        out_shape=jax.ShapeDtypeStruct((B,), jnp.float32),
    )(x_t, gamma, beta, wd, b)


def kernel(sparse_indices, dense_features, emb_tables, gamma, beta, W, b):
    tbl_t = jnp.transpose(emb_tables, (0, 2, 1))
    wemb3 = W[: F * D, 0].reshape(F, D, 1)
    wflat = W[: F * D, 0]
    fidx = jnp.arange(F, dtype=jnp.int32)
    offs = jnp.where(fidx < KSC, fidx * VPS, (fidx - KSC) * VP)
    si3 = sparse_indices.reshape(B // CHUNK_B, CHUNK_B, F)
    idx3d = jnp.transpose(si3, (0, 2, 1)) + offs[None, :, None]
    t_sc = _tc_tail(tbl_t, wemb3, _sc_table(tbl_t, wflat))
    t_tc = _tc_weighted_table(tbl_t, wemb3)
    x_t = jnp.transpose(dense_features, (1, 0))
    d = _tc_stats(
        x_t,
        gamma.reshape(NDENSE, 1),
        beta.reshape(NDENSE, 1),
        W[F * D :, 0].reshape(NDENSE, 1),
        b.reshape(1, 1),
    )
    return _sc_gather(t_sc, t_tc, idx3d, d).reshape(B, 1)
